# baseline (speedup 1.0000x reference)
import functools

import jax
import jax.numpy as jnp
from jax import lax
from jax.experimental import pallas as pl
from jax.experimental.pallas import tpu as pltpu
from jax.experimental.pallas import tpu_sc as plsc

_NUM_WORKERS = 32
_UNROLL = 8


@functools.partial(jax.jit, static_argnums=(2, 3))
def _sc_gather_t(idx_flat, tpack_t, seq, batch):
    npairs, vocab = tpack_t.shape
    emb = 2 * npairs
    mesh = plsc.VectorSubcoreMesh(core_axis_name="c", subcore_axis_name="s")

    @functools.partial(
        pl.kernel,
        out_type=jax.ShapeDtypeStruct((seq, emb, batch), jnp.float32),
        mesh=mesh,
        scratch_types=[
            pltpu.VMEM((vocab,), jnp.int32),
            pltpu.VMEM((batch,), jnp.int32),
            pltpu.VMEM((batch,), jnp.int32),
            pltpu.VMEM((2, batch), jnp.float32),
            pltpu.VMEM((2, batch), jnp.float32),
            *[pltpu.SemaphoreType.DMA for _ in range(4)],
        ],
        compiler_params=pltpu.CompilerParams(needs_layout_passes=False),
    )
    def k(idx_hbm, tab_hbm, out_hbm, tab_v, ib0, ib1, ob0, ob1,
          is0, is1, os0, os1):
        ibs, obs = (ib0, ib1), (ob0, ob1)
        isems, osems = (is0, is1), (os0, os1)
        wid = lax.axis_index("s") * 2 + lax.axis_index("c")

        def idxcopy(s, j):
            return pltpu.make_async_copy(
                idx_hbm.at[pl.ds(s * batch, batch)], ibs[j], isems[j])

        def ostore(s, j):
            return pltpu.make_async_copy(
                obs[j], out_hbm.at[s, pl.ds(2 * wid, 2)], osems[j])

        def compute(j):
            ib, ob = ibs[j], obs[j]

            @plsc.parallel_loop(0, batch // 16, unroll=_UNROLL)
            def _(kk):
                sl = pl.ds(kk * 16, 16)
                packed = plsc.load_gather(tab_v, [ib[sl]])
                hi = jnp.bitwise_and(packed, jnp.int32(-65536))
                lo = packed << 16
                ob[0, sl] = plsc.bitcast(hi, jnp.float32)
                ob[1, sl] = plsc.bitcast(lo, jnp.float32)

        pltpu.sync_copy(tab_hbm.at[wid], tab_v)
        idxcopy(0, 0).start()

        def body(so, carry):
            for j in range(2):
                s = 2 * so + j

                @pl.when(s + 1 < seq)
                def _(s=s, j=j):
                    idxcopy(s + 1, 1 - j).start()

                idxcopy(s, j).wait()

                @pl.when(s >= 2)
                def _(j=j):
                    ostore(0, j).wait()

                compute(j)
                ostore(s, j).start()
            return carry

        lax.fori_loop(0, seq // 2, body, 0)
        for j in range(2):
            ostore(0, j).wait()

    return k(idx_flat, tpack_t)


def kernel(inputs, table):
    batch, seq = inputs.shape
    idx_flat = inputs.T.reshape(-1).astype(jnp.int32)
    tu = jax.lax.bitcast_convert_type(
        table.astype(jnp.bfloat16), jnp.uint16).astype(jnp.uint32)
    packed = (tu[:, 0::2] << 16) | tu[:, 1::2]
    tpack_t = jax.lax.bitcast_convert_type(packed, jnp.int32).T
    out_t = _sc_gather_t(idx_flat, tpack_t, seq, batch)
    return out_t.transpose(2, 0, 1)
